# baseline (speedup 1.0000x reference)
import functools

import numpy as np
import jax
import jax.numpy as jnp
from jax.experimental import pallas as pl
from jax.experimental.pallas import tpu as pltpu


_IMAGENET_MEAN = np.array([0.485, 0.456, 0.406], np.float32).reshape(1, 3, 1, 1)
_IMAGENET_STD = np.array([0.229, 0.224, 0.225], np.float32).reshape(1, 3, 1, 1)

_VMEM = 64 * 1024 * 1024


def _preprocess(img):
    img = (img - _IMAGENET_MEAN) / _IMAGENET_STD
    img = jax.image.resize(
        img, (img.shape[0], img.shape[1], 224, 224), method="bilinear")
    return jnp.transpose(img, (0, 2, 3, 1))


def _stack_w(w, s, kpad=0):
    cin, cout = w.shape[2], w.shape[3]
    w5 = jnp.zeros((3, s + 2, cin, s, cout), jnp.float32)
    for j in range(s):
        w5 = w5.at[:, j:j + 3, :, j, :].set(w)
    wm = w5.reshape(3 * (s + 2) * cin, s * cout)
    if kpad:
        wm = jnp.pad(wm, ((0, kpad), (0, 0)))
    return wm.astype(jnp.bfloat16)


def _tile_b(b, s):
    return jnp.tile(b, s).reshape(1, s * b.shape[0]).astype(jnp.float32)


def _gather_rows(x_ref, r0, nrows, h):
    lo, hi = max(r0, 0), min(r0 + nrows, h)
    tail = x_ref.shape[2:]
    segs = []
    if lo > r0:
        segs.append(jnp.zeros((lo - r0,) + tail, x_ref.dtype))
    segs.append(x_ref[0, lo:hi])
    if r0 + nrows > hi:
        segs.append(jnp.zeros((r0 + nrows - hi,) + tail, x_ref.dtype))
    return jnp.concatenate(segs, axis=0) if len(segs) > 1 else segs[0]


def _load_chunk(x_ref, y0, bh, pool, craw):
    hraw = x_ref.shape[1]
    if not pool:
        return _gather_rows(x_ref, y0 - 1, bh + 2, hraw)
    raw = _gather_rows(x_ref, 2 * (y0 - 1), 2 * (bh + 2), hraw)
    g, cl = raw.shape[1], raw.shape[2]
    r = raw.reshape(bh + 2, 2, g, cl)
    p = jnp.maximum(r[:, 0], r[:, 1])
    npx = cl // craw
    halves = [jnp.maximum(p[..., (2 * q) * craw:(2 * q + 1) * craw],
                          p[..., (2 * q + 1) * craw:(2 * q + 2) * craw])
              for q in range(npx // 2)]
    return jnp.concatenate(halves, axis=-1) if len(halves) > 1 else halves[0]


def _patches(v3, bh, G, cin):
    cl = v3.shape[-1]
    zc = jnp.zeros((bh, 1, cin), v3.dtype)
    parts = []
    for dy in range(3):
        v = v3[dy:dy + bh]
        pa = jnp.concatenate([zc, v[:, 0:G - 1, cl - cin:]], axis=1)
        pc = jnp.concatenate([v[:, 1:G, 0:cin], zc], axis=1)
        parts += [pa, v, pc]
    return jnp.concatenate(parts, axis=-1).reshape(bh * G, -1)


def _conv_body(x_ref, w_ref, b_ref, o_ref, *, cin, G, bh, nch, pool, craw):
    for ch in range(nch):
        y0 = ch * bh
        v3 = _load_chunk(x_ref, y0, bh, pool, craw)
        z = _patches(v3, bh, G, cin)
        acc = jnp.dot(z, w_ref[...], preferred_element_type=jnp.float32)
        acc = jnp.maximum(acc + b_ref[...], 0.0)
        o_ref[0, y0:y0 + bh] = acc.reshape(bh, G, -1).astype(o_ref.dtype)


def _mm_body(x_ref, w_ref, b_ref, o_ref, *, G, bh, nch):
    k = x_ref.shape[-1]
    for ch in range(nch):
        y0 = ch * bh
        z = x_ref[0, y0:y0 + bh].reshape(bh * G, k)
        acc = jnp.dot(z, w_ref[...], preferred_element_type=jnp.float32)
        acc = jnp.maximum(acc + b_ref[...], 0.0)
        o_ref[0, y0:y0 + bh] = acc.reshape(bh, G, -1).astype(o_ref.dtype)


def _conv_l1_body(x1_ref, x2_ref, w_ref, b_ref, o_ref, *, cin, G, bh, nch):
    tot = jnp.zeros((1, 256), jnp.float32)
    m = bh * G
    for ch in range(nch):
        y0 = ch * bh
        z1 = _patches(_load_chunk(x1_ref, y0, bh, False, cin), bh, G, cin)
        z2 = _patches(_load_chunk(x2_ref, y0, bh, False, cin), bh, G, cin)
        z = jnp.concatenate([z1, z2], axis=0)
        acc = jnp.dot(z, w_ref[...], preferred_element_type=jnp.float32)
        acc = jnp.maximum(acc + b_ref[...], 0.0)
        d = jnp.abs(acc[:m] - acc[m:])
        tot = tot + jnp.sum(d, axis=0, keepdims=True)
    o_ref[...] = tot.reshape(1, 1, 256)


def _l1_body(x_ref, y_ref, o_ref):
    d = jnp.abs(x_ref[...] - y_ref[...])
    o_ref[...] = jnp.sum(d, axis=0, keepdims=True).reshape(1, 1, 256)


def _conv(x, wst, bt, *, H, G, s, cin, cout, bh, pool=False, craw=0):
    n = x.shape[0]
    body = functools.partial(_conv_body, cin=cin, G=G, bh=bh, nch=H // bh,
                             pool=pool, craw=craw)
    return pl.pallas_call(
        body,
        out_shape=jax.ShapeDtypeStruct((n, H, G, s * cout), jnp.bfloat16),
        grid=(n,),
        in_specs=[
            pl.BlockSpec((1,) + x.shape[1:], lambda i: (i, 0, 0, 0)),
            pl.BlockSpec(wst.shape, lambda i: (0, 0)),
            pl.BlockSpec((1, s * cout), lambda i: (0, 0)),
        ],
        out_specs=pl.BlockSpec((1, H, G, s * cout), lambda i: (i, 0, 0, 0)),
        compiler_params=pltpu.CompilerParams(
            dimension_semantics=("parallel",), vmem_limit_bytes=_VMEM),
    )(x, wst, bt)


def _mm(p, wst, bt, *, H, G, bh, nout):
    n = p.shape[0]
    body = functools.partial(_mm_body, G=G, bh=bh, nch=H // bh)
    return pl.pallas_call(
        body,
        out_shape=jax.ShapeDtypeStruct((n, H, G, nout), jnp.bfloat16),
        grid=(n,),
        in_specs=[
            pl.BlockSpec((1,) + p.shape[1:], lambda i: (i, 0, 0, 0)),
            pl.BlockSpec(wst.shape, lambda i: (0, 0)),
            pl.BlockSpec((1, nout), lambda i: (0, 0)),
        ],
        out_specs=pl.BlockSpec((1, H, G, nout), lambda i: (i, 0, 0, 0)),
        compiler_params=pltpu.CompilerParams(
            dimension_semantics=("parallel",), vmem_limit_bytes=_VMEM),
    )(p, wst, bt)


def _conv_l1(x, wst, bt, *, H, G, cin, bh, npair):
    body = functools.partial(_conv_l1_body, cin=cin, G=G, bh=bh, nch=H // bh)
    blk = (1,) + x.shape[1:]
    return pl.pallas_call(
        body,
        out_shape=jax.ShapeDtypeStruct((npair, 1, 256), jnp.float32),
        grid=(npair,),
        in_specs=[
            pl.BlockSpec(blk, lambda i: (i, 0, 0, 0)),
            pl.BlockSpec(blk, lambda i: (i + npair, 0, 0, 0)),
            pl.BlockSpec(wst.shape, lambda i: (0, 0)),
            pl.BlockSpec((1, 256), lambda i: (0, 0)),
        ],
        out_specs=pl.BlockSpec((1, 1, 256), lambda i: (i, 0, 0)),
        compiler_params=pltpu.CompilerParams(
            dimension_semantics=("parallel",), vmem_limit_bytes=_VMEM),
    )(x, x, wst, bt)


def _l1_mean(x, y):
    rows = x.size // 256
    nblk = 8
    x2 = x.reshape(rows, 256)
    y2 = y.reshape(rows, 256)
    part = pl.pallas_call(
        _l1_body,
        out_shape=jax.ShapeDtypeStruct((nblk, 1, 256), jnp.float32),
        grid=(nblk,),
        in_specs=[
            pl.BlockSpec((rows // nblk, 256), lambda i: (i, 0)),
            pl.BlockSpec((rows // nblk, 256), lambda i: (i, 0)),
        ],
        out_specs=pl.BlockSpec((1, 1, 256), lambda i: (i, 0, 0)),
        compiler_params=pltpu.CompilerParams(
            dimension_semantics=("parallel",)),
    )(x2, y2)
    return jnp.sum(part) / x.size


def kernel(out1, gt1,
           w_0_0, b_0_0, w_0_1, b_0_1,
           w_1_0, b_1_0, w_1_1, b_1_1,
           w_2_0, b_2_0, w_2_1, b_2_1, w_2_2, b_2_2):
    pixel_l1 = _l1_mean(out1.astype(jnp.float32), gt1.astype(jnp.float32))

    xy = jnp.concatenate([_preprocess(out1), _preprocess(gt1)], axis=0)

    xp = jnp.pad(xy, ((0, 0), (1, 1), (1, 3), (0, 0))).astype(jnp.bfloat16)
    v = xp.reshape(32, 226, 57, 12)
    parts = []
    for dy in range(3):
        vd = v[:, dy:dy + 224]
        parts += [vd[:, :, 0:56, :], vd[:, :, 1:57, 0:6]]
    parts.append(jnp.zeros((32, 224, 56, 74), jnp.bfloat16))
    p0 = jnp.concatenate(parts, axis=-1)
    a = _mm(p0, _stack_w(w_0_0, 4, kpad=74), _tile_b(b_0_0, 4),
            H=224, G=56, bh=56, nout=256)

    a = _conv(a, _stack_w(w_0_1, 4), _tile_b(b_0_1, 4),
              H=224, G=56, s=4, cin=64, cout=64, bh=56)
    a = _conv(a, _stack_w(w_1_0, 2), _tile_b(b_1_0, 2),
              H=112, G=56, s=2, cin=64, cout=128, bh=56, pool=True, craw=64)
    a = _conv(a, _stack_w(w_1_1, 2), _tile_b(b_1_1, 2),
              H=112, G=56, s=2, cin=128, cout=128, bh=56)
    a = _conv(a, _stack_w(w_2_0, 1), _tile_b(b_2_0, 1),
              H=56, G=56, s=1, cin=128, cout=256, bh=56, pool=True, craw=128)
    a = _conv(a, _stack_w(w_2_1, 1), _tile_b(b_2_1, 1),
              H=56, G=56, s=1, cin=256, cout=256, bh=28)

    perc_part = _conv_l1(a, _stack_w(w_2_2, 1), _tile_b(b_2_2, 1),
                         H=56, G=56, cin=256, bh=28, npair=16)
    perceptual = jnp.sum(perc_part) / np.float32(16 * 56 * 56 * 256)
    return perceptual + pixel_l1


# speedup vs baseline: 1.6776x; 1.6776x over previous
import functools

import numpy as np
import jax
import jax.numpy as jnp
from jax.experimental import pallas as pl
from jax.experimental.pallas import tpu as pltpu


_IMAGENET_MEAN = np.array([0.485, 0.456, 0.406], np.float32).reshape(1, 3, 1, 1)
_IMAGENET_STD = np.array([0.229, 0.224, 0.225], np.float32).reshape(1, 3, 1, 1)

_VMEM = 64 * 1024 * 1024


def _preprocess(img):
    img = (img - _IMAGENET_MEAN) / _IMAGENET_STD
    img = jax.image.resize(
        img, (img.shape[0], img.shape[1], 224, 224), method="bilinear")
    return jnp.transpose(img, (0, 2, 3, 1))


def _stack_w(w, s, kpad=0):
    cin, cout = w.shape[2], w.shape[3]
    w5 = jnp.zeros((3, s + 2, cin, s, cout), jnp.float32)
    for j in range(s):
        w5 = w5.at[:, j:j + 3, :, j, :].set(w)
    wm = w5.reshape(3 * (s + 2) * cin, s * cout)
    if kpad:
        wm = jnp.pad(wm, ((0, kpad), (0, 0)))
    return wm.astype(jnp.bfloat16)


def _tile_b(b, s):
    return jnp.tile(b, s).reshape(1, s * b.shape[0]).astype(jnp.float32)


def _gather_rows(x_ref, r0, nrows, h):
    lo, hi = max(r0, 0), min(r0 + nrows, h)
    tail = x_ref.shape[2:]
    segs = []
    if lo > r0:
        segs.append(jnp.zeros((lo - r0,) + tail, x_ref.dtype))
    segs.append(x_ref[0, lo:hi])
    if r0 + nrows > hi:
        segs.append(jnp.zeros((r0 + nrows - hi,) + tail, x_ref.dtype))
    return jnp.concatenate(segs, axis=0) if len(segs) > 1 else segs[0]


def _load_chunk(x_ref, y0, bh, pool, craw):
    hraw = x_ref.shape[1]
    if not pool:
        return _gather_rows(x_ref, y0 - 1, bh + 2, hraw)
    raw = _gather_rows(x_ref, 2 * (y0 - 1), 2 * (bh + 2), hraw)
    g, cl = raw.shape[1], raw.shape[2]
    r = raw.reshape(bh + 2, 2, g, cl)
    p = jnp.maximum(r[:, 0], r[:, 1])
    npx = cl // craw
    halves = [jnp.maximum(p[..., (2 * q) * craw:(2 * q + 1) * craw],
                          p[..., (2 * q + 1) * craw:(2 * q + 2) * craw])
              for q in range(npx // 2)]
    return jnp.concatenate(halves, axis=-1) if len(halves) > 1 else halves[0]


def _patches(v3, bh, G, cin):
    cl = v3.shape[-1]
    zc = jnp.zeros((bh, 1, cin), v3.dtype)
    parts = []
    for dy in range(3):
        v = v3[dy:dy + bh]
        pa = jnp.concatenate([zc, v[:, 0:G - 1, cl - cin:]], axis=1)
        pc = jnp.concatenate([v[:, 1:G, 0:cin], zc], axis=1)
        parts += [pa, v, pc]
    return jnp.concatenate(parts, axis=-1).reshape(bh * G, -1)


def _conv_body(x_ref, w_ref, b_ref, o_ref, *, cin, G, bh, nch, pool, craw):
    for ch in range(nch):
        y0 = ch * bh
        v3 = _load_chunk(x_ref, y0, bh, pool, craw)
        z = _patches(v3, bh, G, cin)
        acc = jnp.dot(z, w_ref[...], preferred_element_type=jnp.float32)
        acc = jnp.maximum(acc + b_ref[...], 0.0)
        o_ref[0, y0:y0 + bh] = acc.reshape(bh, G, -1).astype(o_ref.dtype)


def _mm_body(x_ref, w_ref, b_ref, o_ref, *, G, bh, nch):
    va = x_ref[0].reshape(x_ref.shape[1], G + 1, 12)
    for ch in range(nch):
        y0 = ch * bh
        parts = []
        for dy in range(3):
            v = va[y0 + dy:y0 + dy + bh]
            parts += [v[:, 0:G, :], v[:, 1:G + 1, 0:6]]
        z = jnp.concatenate(parts, axis=-1).reshape(bh * G, 54)
        acc = jnp.dot(z, w_ref[...], preferred_element_type=jnp.float32)
        acc = jnp.maximum(acc + b_ref[...], 0.0)
        o_ref[0, y0:y0 + bh] = acc.reshape(bh, G, -1).astype(o_ref.dtype)


def _conv_l1_body(x1_ref, x2_ref, w_ref, b_ref, o_ref, *, cin, G, bh, nch):
    tot = jnp.zeros((1, 256), jnp.float32)
    m = bh * G
    for ch in range(nch):
        y0 = ch * bh
        z1 = _patches(_load_chunk(x1_ref, y0, bh, False, cin), bh, G, cin)
        z2 = _patches(_load_chunk(x2_ref, y0, bh, False, cin), bh, G, cin)
        z = jnp.concatenate([z1, z2], axis=0)
        acc = jnp.dot(z, w_ref[...], preferred_element_type=jnp.float32)
        acc = jnp.maximum(acc + b_ref[...], 0.0)
        d = jnp.abs(acc[:m] - acc[m:])
        tot = tot + jnp.sum(d, axis=0, keepdims=True)
    o_ref[...] = tot.reshape(1, 1, 256)


def _l1_body(x_ref, y_ref, o_ref):
    d = jnp.abs(x_ref[...] - y_ref[...])
    o_ref[...] = jnp.sum(d, axis=0, keepdims=True).reshape(1, 1, 256)


def _conv(x, wst, bt, *, H, G, s, cin, cout, bh, pool=False, craw=0):
    n = x.shape[0]
    body = functools.partial(_conv_body, cin=cin, G=G, bh=bh, nch=H // bh,
                             pool=pool, craw=craw)
    return pl.pallas_call(
        body,
        out_shape=jax.ShapeDtypeStruct((n, H, G, s * cout), jnp.bfloat16),
        grid=(n,),
        in_specs=[
            pl.BlockSpec((1,) + x.shape[1:], lambda i: (i, 0, 0, 0)),
            pl.BlockSpec(wst.shape, lambda i: (0, 0)),
            pl.BlockSpec((1, s * cout), lambda i: (0, 0)),
        ],
        out_specs=pl.BlockSpec((1, H, G, s * cout), lambda i: (i, 0, 0, 0)),
        compiler_params=pltpu.CompilerParams(
            dimension_semantics=("parallel",), vmem_limit_bytes=_VMEM),
    )(x, wst, bt)


def _mm(p, wst, bt, *, H, G, bh, nout):
    n = p.shape[0]
    body = functools.partial(_mm_body, G=G, bh=bh, nch=H // bh)
    return pl.pallas_call(
        body,
        out_shape=jax.ShapeDtypeStruct((n, H, G, nout), jnp.bfloat16),
        grid=(n,),
        in_specs=[
            pl.BlockSpec((1,) + p.shape[1:], lambda i: (i, 0, 0)),
            pl.BlockSpec(wst.shape, lambda i: (0, 0)),
            pl.BlockSpec((1, nout), lambda i: (0, 0)),
        ],
        out_specs=pl.BlockSpec((1, H, G, nout), lambda i: (i, 0, 0, 0)),
        compiler_params=pltpu.CompilerParams(
            dimension_semantics=("parallel",), vmem_limit_bytes=_VMEM),
    )(p, wst, bt)


def _conv_l1(x, wst, bt, *, H, G, cin, bh, npair):
    body = functools.partial(_conv_l1_body, cin=cin, G=G, bh=bh, nch=H // bh)
    blk = (1,) + x.shape[1:]
    return pl.pallas_call(
        body,
        out_shape=jax.ShapeDtypeStruct((npair, 1, 256), jnp.float32),
        grid=(npair,),
        in_specs=[
            pl.BlockSpec(blk, lambda i: (i, 0, 0, 0)),
            pl.BlockSpec(blk, lambda i: (i + npair, 0, 0, 0)),
            pl.BlockSpec(wst.shape, lambda i: (0, 0)),
            pl.BlockSpec((1, 256), lambda i: (0, 0)),
        ],
        out_specs=pl.BlockSpec((1, 1, 256), lambda i: (i, 0, 0)),
        compiler_params=pltpu.CompilerParams(
            dimension_semantics=("parallel",), vmem_limit_bytes=_VMEM),
    )(x, x, wst, bt)


def _l1_mean(x, y):
    rows = x.size // 256
    nblk = 8
    x2 = x.reshape(rows, 256)
    y2 = y.reshape(rows, 256)
    part = pl.pallas_call(
        _l1_body,
        out_shape=jax.ShapeDtypeStruct((nblk, 1, 256), jnp.float32),
        grid=(nblk,),
        in_specs=[
            pl.BlockSpec((rows // nblk, 256), lambda i: (i, 0)),
            pl.BlockSpec((rows // nblk, 256), lambda i: (i, 0)),
        ],
        out_specs=pl.BlockSpec((1, 1, 256), lambda i: (i, 0, 0)),
        compiler_params=pltpu.CompilerParams(
            dimension_semantics=("parallel",)),
    )(x2, y2)
    return jnp.sum(part) / x.size


def kernel(out1, gt1,
           w_0_0, b_0_0, w_0_1, b_0_1,
           w_1_0, b_1_0, w_1_1, b_1_1,
           w_2_0, b_2_0, w_2_1, b_2_1, w_2_2, b_2_2):
    pixel_l1 = _l1_mean(out1.astype(jnp.float32), gt1.astype(jnp.float32))

    xy = jnp.concatenate([_preprocess(out1), _preprocess(gt1)], axis=0)

    xf = xy.reshape(32, 224, 672)
    p0 = jnp.pad(xf, ((0, 0), (1, 1), (3, 9))).astype(jnp.bfloat16)
    a = _mm(p0, _stack_w(w_0_0, 4), _tile_b(b_0_0, 4),
            H=224, G=56, bh=56, nout=256)

    a = _conv(a, _stack_w(w_0_1, 4), _tile_b(b_0_1, 4),
              H=224, G=56, s=4, cin=64, cout=64, bh=56)
    a = _conv(a, _stack_w(w_1_0, 2), _tile_b(b_1_0, 2),
              H=112, G=56, s=2, cin=64, cout=128, bh=56, pool=True, craw=64)
    a = _conv(a, _stack_w(w_1_1, 2), _tile_b(b_1_1, 2),
              H=112, G=56, s=2, cin=128, cout=128, bh=56)
    a = _conv(a, _stack_w(w_2_0, 1), _tile_b(b_2_0, 1),
              H=56, G=56, s=1, cin=128, cout=256, bh=56, pool=True, craw=128)
    a = _conv(a, _stack_w(w_2_1, 1), _tile_b(b_2_1, 1),
              H=56, G=56, s=1, cin=256, cout=256, bh=28)

    perc_part = _conv_l1(a, _stack_w(w_2_2, 1), _tile_b(b_2_2, 1),
                         H=56, G=56, cin=256, bh=28, npair=16)
    perceptual = jnp.sum(perc_part) / np.float32(16 * 56 * 56 * 256)
    return perceptual + pixel_l1


# speedup vs baseline: 1.7008x; 1.0138x over previous
import functools

import numpy as np
import jax
import jax.numpy as jnp
from jax.experimental import pallas as pl
from jax.experimental.pallas import tpu as pltpu


_IMAGENET_MEAN = np.array([0.485, 0.456, 0.406], np.float32).reshape(1, 3, 1, 1)
_IMAGENET_STD = np.array([0.229, 0.224, 0.225], np.float32).reshape(1, 3, 1, 1)

_VMEM = 64 * 1024 * 1024


def _preprocess(img):
    img = (img - _IMAGENET_MEAN) / _IMAGENET_STD
    img = jax.image.resize(
        img, (img.shape[0], img.shape[1], 224, 224), method="bilinear")
    return jnp.transpose(img, (0, 2, 3, 1))


def _stack_w(w, s):
    cin, cout = w.shape[2], w.shape[3]
    w5 = jnp.zeros((3, s + 2, cin, s, cout), jnp.float32)
    for j in range(s):
        w5 = w5.at[:, j:j + 3, :, j, :].set(w)
    return w5.reshape(3 * (s + 2) * cin, s * cout).astype(jnp.bfloat16)


def _tile_b(b, s):
    return jnp.tile(b, s).reshape(1, s * b.shape[0]).astype(jnp.float32)


def _gather_rows(x_ref, r0, nrows, h):
    lo, hi = max(r0, 0), min(r0 + nrows, h)
    tail = x_ref.shape[2:]
    segs = []
    if lo > r0:
        segs.append(jnp.zeros((lo - r0,) + tail, x_ref.dtype))
    segs.append(x_ref[0, lo:hi])
    if r0 + nrows > hi:
        segs.append(jnp.zeros((r0 + nrows - hi,) + tail, x_ref.dtype))
    return jnp.concatenate(segs, axis=0) if len(segs) > 1 else segs[0]


def _load_chunk(x_ref, y0, bh, pool, craw):
    hraw = x_ref.shape[1]
    if not pool:
        return _gather_rows(x_ref, y0 - 1, bh + 2, hraw)
    raw = _gather_rows(x_ref, 2 * (y0 - 1), 2 * (bh + 2), hraw)
    g, cl = raw.shape[1], raw.shape[2]
    r = raw.reshape(bh + 2, 2, g, cl)
    p = jnp.maximum(r[:, 0], r[:, 1])
    npx = cl // craw
    halves = [jnp.maximum(p[..., (2 * q) * craw:(2 * q + 1) * craw],
                          p[..., (2 * q + 1) * craw:(2 * q + 2) * craw])
              for q in range(npx // 2)]
    return jnp.concatenate(halves, axis=-1) if len(halves) > 1 else halves[0]


def _patches(v3, bh, G, cin):
    cl = v3.shape[-1]
    zc = jnp.zeros((bh, 1, cin), v3.dtype)
    parts = []
    for dy in range(3):
        v = v3[dy:dy + bh]
        pa = jnp.concatenate([zc, v[:, 0:G - 1, cl - cin:]], axis=1)
        pc = jnp.concatenate([v[:, 1:G, 0:cin], zc], axis=1)
        parts += [pa, v, pc]
    return jnp.concatenate(parts, axis=-1).reshape(bh * G, -1)


def _conv_into(x_ref, w_ref, b_ref, o_ref, *, cin, G, bh, nch, pool, craw):
    for ch in range(nch):
        y0 = ch * bh
        v3 = _load_chunk(x_ref, y0, bh, pool, craw)
        z = _patches(v3, bh, G, cin)
        acc = jnp.dot(z, w_ref[...], preferred_element_type=jnp.float32)
        acc = jnp.maximum(acc + b_ref[...], 0.0)
        o_ref[0, y0:y0 + bh] = acc.reshape(bh, G, -1).astype(o_ref.dtype)


def _block0_body(x_ref, w0_ref, b0_ref, w1_ref, b1_ref, o_ref, a0_ref):
    va = x_ref[0].reshape(226, 57, 12)
    for ch in range(4):
        y0 = ch * 56
        parts = []
        for dy in range(3):
            v = va[y0 + dy:y0 + dy + 56]
            parts += [v[:, 0:56, :], v[:, 1:57, 0:6]]
        z = jnp.concatenate(parts, axis=-1).reshape(56 * 56, 54)
        acc = jnp.dot(z, w0_ref[...], preferred_element_type=jnp.float32)
        acc = jnp.maximum(acc + b0_ref[...], 0.0)
        a0_ref[0, y0:y0 + 56] = acc.reshape(56, 56, 256).astype(a0_ref.dtype)
    _conv_into(a0_ref, w1_ref, b1_ref, o_ref,
               cin=64, G=56, bh=56, nch=4, pool=False, craw=0)


def _block1_body(x_ref, w0_ref, b0_ref, w1_ref, b1_ref, o_ref, a2_ref):
    _conv_into(x_ref, w0_ref, b0_ref, a2_ref,
               cin=64, G=56, bh=56, nch=2, pool=True, craw=64)
    _conv_into(a2_ref, w1_ref, b1_ref, o_ref,
               cin=128, G=56, bh=56, nch=2, pool=False, craw=0)


def _block2_body(x1_ref, x2_ref, w0_ref, b0_ref, w1_ref, b1_ref,
                 w2_ref, b2_ref, o_ref, a4a, a4b, a5a, a5b):
    for xr, a4, a5 in ((x1_ref, a4a, a5a), (x2_ref, a4b, a5b)):
        _conv_into(xr, w0_ref, b0_ref, a4,
                   cin=128, G=56, bh=56, nch=1, pool=True, craw=128)
        _conv_into(a4, w1_ref, b1_ref, a5,
                   cin=256, G=56, bh=28, nch=2, pool=False, craw=0)
    tot = jnp.zeros((1, 256), jnp.float32)
    m = 28 * 56
    for ch in range(2):
        y0 = ch * 28
        z1 = _patches(_load_chunk(a5a, y0, 28, False, 0), 28, 56, 256)
        z2 = _patches(_load_chunk(a5b, y0, 28, False, 0), 28, 56, 256)
        z = jnp.concatenate([z1, z2], axis=0)
        acc = jnp.dot(z, w2_ref[...], preferred_element_type=jnp.float32)
        acc = jnp.maximum(acc + b2_ref[...], 0.0)
        d = jnp.abs(acc[:m] - acc[m:])
        tot = tot + jnp.sum(d, axis=0, keepdims=True)
    o_ref[...] = tot.reshape(1, 1, 256)


def _l1_body(x_ref, y_ref, o_ref):
    d = jnp.abs(x_ref[...] - y_ref[...])
    o_ref[...] = jnp.sum(d, axis=0, keepdims=True).reshape(1, 1, 256)


def _block0(p0, w0, b0, w1, b1):
    return pl.pallas_call(
        _block0_body,
        out_shape=jax.ShapeDtypeStruct((32, 224, 56, 256), jnp.bfloat16),
        grid=(32,),
        in_specs=[
            pl.BlockSpec((1, 226, 684), lambda i: (i, 0, 0)),
            pl.BlockSpec(w0.shape, lambda i: (0, 0)),
            pl.BlockSpec((1, 256), lambda i: (0, 0)),
            pl.BlockSpec(w1.shape, lambda i: (0, 0)),
            pl.BlockSpec((1, 256), lambda i: (0, 0)),
        ],
        out_specs=pl.BlockSpec((1, 224, 56, 256), lambda i: (i, 0, 0, 0)),
        scratch_shapes=[pltpu.VMEM((1, 224, 56, 256), jnp.bfloat16)],
        compiler_params=pltpu.CompilerParams(
            dimension_semantics=("parallel",), vmem_limit_bytes=_VMEM),
    )(p0, w0, b0, w1, b1)


def _block1(a1, w0, b0, w1, b1):
    return pl.pallas_call(
        _block1_body,
        out_shape=jax.ShapeDtypeStruct((32, 112, 56, 256), jnp.bfloat16),
        grid=(32,),
        in_specs=[
            pl.BlockSpec((1, 224, 56, 256), lambda i: (i, 0, 0, 0)),
            pl.BlockSpec(w0.shape, lambda i: (0, 0)),
            pl.BlockSpec((1, 256), lambda i: (0, 0)),
            pl.BlockSpec(w1.shape, lambda i: (0, 0)),
            pl.BlockSpec((1, 256), lambda i: (0, 0)),
        ],
        out_specs=pl.BlockSpec((1, 112, 56, 256), lambda i: (i, 0, 0, 0)),
        scratch_shapes=[pltpu.VMEM((1, 112, 56, 256), jnp.bfloat16)],
        compiler_params=pltpu.CompilerParams(
            dimension_semantics=("parallel",), vmem_limit_bytes=_VMEM),
    )(a1, w0, b0, w1, b1)


def _block2(a3, w0, b0, w1, b1, w2, b2):
    blk = (1, 112, 56, 256)
    return pl.pallas_call(
        _block2_body,
        out_shape=jax.ShapeDtypeStruct((16, 1, 256), jnp.float32),
        grid=(16,),
        in_specs=[
            pl.BlockSpec(blk, lambda i: (i, 0, 0, 0)),
            pl.BlockSpec(blk, lambda i: (i + 16, 0, 0, 0)),
            pl.BlockSpec(w0.shape, lambda i: (0, 0)),
            pl.BlockSpec((1, 256), lambda i: (0, 0)),
            pl.BlockSpec(w1.shape, lambda i: (0, 0)),
            pl.BlockSpec((1, 256), lambda i: (0, 0)),
            pl.BlockSpec(w2.shape, lambda i: (0, 0)),
            pl.BlockSpec((1, 256), lambda i: (0, 0)),
        ],
        out_specs=pl.BlockSpec((1, 1, 256), lambda i: (i, 0, 0)),
        scratch_shapes=[pltpu.VMEM((1, 56, 56, 256), jnp.bfloat16)
                        for _ in range(4)],
        compiler_params=pltpu.CompilerParams(
            dimension_semantics=("parallel",), vmem_limit_bytes=_VMEM),
    )(a3, a3, w0, b0, w1, b1, w2, b2)


def _l1_mean(x, y):
    rows = x.size // 256
    nblk = 8
    x2 = x.reshape(rows, 256)
    y2 = y.reshape(rows, 256)
    part = pl.pallas_call(
        _l1_body,
        out_shape=jax.ShapeDtypeStruct((nblk, 1, 256), jnp.float32),
        grid=(nblk,),
        in_specs=[
            pl.BlockSpec((rows // nblk, 256), lambda i: (i, 0)),
            pl.BlockSpec((rows // nblk, 256), lambda i: (i, 0)),
        ],
        out_specs=pl.BlockSpec((1, 1, 256), lambda i: (i, 0, 0)),
        compiler_params=pltpu.CompilerParams(
            dimension_semantics=("parallel",)),
    )(x2, y2)
    return jnp.sum(part) / x.size


def kernel(out1, gt1,
           w_0_0, b_0_0, w_0_1, b_0_1,
           w_1_0, b_1_0, w_1_1, b_1_1,
           w_2_0, b_2_0, w_2_1, b_2_1, w_2_2, b_2_2):
    pixel_l1 = _l1_mean(out1.astype(jnp.float32), gt1.astype(jnp.float32))

    xy = jnp.concatenate([_preprocess(out1), _preprocess(gt1)], axis=0)
    p0 = jnp.pad(xy.reshape(32, 224, 672),
                 ((0, 0), (1, 1), (3, 9))).astype(jnp.bfloat16)

    a1 = _block0(p0, _stack_w(w_0_0, 4), _tile_b(b_0_0, 4),
                 _stack_w(w_0_1, 4), _tile_b(b_0_1, 4))
    a3 = _block1(a1, _stack_w(w_1_0, 2), _tile_b(b_1_0, 2),
                 _stack_w(w_1_1, 2), _tile_b(b_1_1, 2))
    perc_part = _block2(a3, _stack_w(w_2_0, 1), _tile_b(b_2_0, 1),
                        _stack_w(w_2_1, 1), _tile_b(b_2_1, 1),
                        _stack_w(w_2_2, 1), _tile_b(b_2_2, 1))
    perceptual = jnp.sum(perc_part) / np.float32(16 * 56 * 56 * 256)
    return perceptual + pixel_l1
